# baseline (speedup 1.0000x reference)
import functools

import jax
import jax.numpy as jnp
from jax import lax
from jax.experimental import pallas as pl
from jax.experimental.pallas import tpu as pltpu
from jax.experimental.pallas import tpu_sc as plsc

N = 10000
NP = 10240
D = 128
E = 320000
EP = 327680
NC = 2
NS = 16
RPT = EP // (NC * NS * 128)
WPT = NP // NS


def _make_sc_agg(with_deg):
  out_type = [jax.ShapeDtypeStruct((NC, N, D), jnp.float32)]
  if with_deg:
    out_type.append(jax.ShapeDtypeStruct((NC, NP), jnp.float32))
  mesh = plsc.VectorSubcoreMesh(
      core_axis_name="c", subcore_axis_name="s",
      num_cores=NC, num_subcores=NS)
  scratch = [
      pltpu.VMEM((2, 8, 128), jnp.int32),
      pltpu.VMEM((2, 8, 128), jnp.int32),
      pltpu.VMEM((4, 64, D), jnp.float32),
      pltpu.VMEM((64,), jnp.float32),
      pltpu.VMEM((WPT,), jnp.float32),
      pltpu.VMEM_SHARED((NP, D), jnp.float32),
      pltpu.VMEM_SHARED((NP,), jnp.float32),
      [pltpu.SemaphoreType.DMA] * 4,
      [pltpu.SemaphoreType.DMA] * 4,
      [pltpu.SemaphoreType.DMA] * 2,
  ]

  @functools.partial(pl.kernel, out_type=out_type, mesh=mesh,
                     scratch_types=scratch)
  def sc_agg(m_hbm, src_hbm, dst_hbm, *rest):
    if with_deg:
      (p_out, deg_out, src_v, dst_v, rows_v, ones_v, zrow_v, acc_sh, deg_sh,
       gsem, ssem, isem) = rest
    else:
      (p_out, src_v, dst_v, rows_v, ones_v, zrow_v, acc_sh, deg_sh,
       gsem, ssem, isem) = rest
    c = lax.axis_index("c")
    s = lax.axis_index("s")
    zero16 = jnp.zeros((16,), jnp.float32)

    def zrows(r, carry):
      for b in range(2):
        for q in range(D // 16):
          rows_v[b, r, pl.ds(q * 16, 16)] = zero16
      return carry
    lax.fori_loop(0, 64, zrows, 0)
    if with_deg:
      ones16 = jnp.ones((16,), jnp.float32)
      for q in range(64 // 16):
        ones_v[pl.ds(q * 16, 16)] = ones16
      def zdeg(i, carry):
        zrow_v[pl.ds(i * 16, 16)] = zero16
        return carry
      lax.fori_loop(0, WPT // 16, zdeg, 0)

    row0 = s * WPT
    def zacc(b, carry):
      pltpu.async_copy(rows_v.at[0], acc_sh.at[pl.ds(row0 + b * 128, 64)],
                       gsem[0])
      pltpu.async_copy(rows_v.at[1], acc_sh.at[pl.ds(row0 + b * 128 + 64, 64)],
                       gsem[1])
      return carry
    lax.fori_loop(0, WPT // 128, zacc, 0)
    def zacc_drain(b, carry):
      pltpu.make_async_copy(rows_v.at[0], acc_sh.at[pl.ds(row0, 64)],
                            gsem[0]).wait()
      pltpu.make_async_copy(rows_v.at[1], acc_sh.at[pl.ds(row0, 64)],
                            gsem[1]).wait()
      return carry
    lax.fori_loop(0, WPT // 128, zacc_drain, 0)
    if with_deg:
      pltpu.sync_copy(zrow_v, deg_sh.at[pl.ds(row0, WPT)])
    plsc.subcore_barrier()

    w = c * NS + s

    def idx_issue(t_next, dstset):
      start = pl.multiple_of(8 * t_next, 8)
      pltpu.async_copy(src_hbm.at[w, pl.ds(start, 8)], src_v.at[dstset],
                       isem[dstset])
      pltpu.async_copy(dst_hbm.at[w, pl.ds(start, 8)], dst_v.at[dstset],
                       isem[dstset])
    def idx_drain(dstset):
      pltpu.make_async_copy(src_hbm.at[w, pl.ds(0, 8)], src_v.at[dstset],
                            isem[dstset]).wait()
      pltpu.make_async_copy(dst_hbm.at[w, pl.ds(0, 8)], dst_v.at[dstset],
                            isem[dstset]).wait()
    def g_issue(P, u, b):
      pltpu.async_copy(
          m_hbm.at[src_v.at[P, u // 2, pl.ds((u % 2) * 64, 64)]],
          rows_v.at[b], gsem[b])
    def g_drain(b):
      pltpu.make_async_copy(m_hbm.at[src_v.at[0, 0, pl.ds(0, 64)]],
                            rows_v.at[b], gsem[b]).wait()
    def s_issue(P, u, b):
      pltpu.async_copy(rows_v.at[b],
                       acc_sh.at[dst_v.at[P, u // 2, pl.ds((u % 2) * 64, 64)]],
                       ssem[b], add=True)
      if with_deg:
        pltpu.sync_copy(ones_v,
                        deg_sh.at[dst_v.at[P, u // 2, pl.ds((u % 2) * 64, 64)]],
                        add=True)
    def s_drain(b):
      pltpu.make_async_copy(rows_v.at[b], acc_sh.at[dst_v.at[0, 0, pl.ds(0, 64)]],
                            ssem[b]).wait()

    def block(t_next, P, first, last):
      for u in range(16):
        b = u % 4
        b2 = (u + 2) % 4
        g_drain(b)
        s_issue(P, u, b)
        if u == 2 and not last:
          idx_issue(t_next, 1 - P)
        if not (first and u < 2):
          s_drain(b2)
        if u == 13 and not last:
          idx_drain(1 - P)
        if u < 14:
          g_issue(P, u + 2, b2)
        elif not last:
          g_issue(1 - P, u - 14, b2)

    idx_issue(jnp.int32(0), 0)
    idx_drain(0)
    g_issue(0, 0, 0)
    g_issue(0, 1, 1)
    block(jnp.int32(1), 0, first=True, last=False)

    def pair_body(p, carry):
      block(2 * p + 2, 1, first=False, last=False)
      block(2 * p + 3, 0, first=False, last=False)
      return carry
    lax.fori_loop(0, 4, pair_body, 0)

    block(jnp.int32(0), 1, first=False, last=True)
    s_drain(2)
    s_drain(3)
    plsc.subcore_barrier()

    @pl.when(s < NS - 1)
    def _():
      pltpu.sync_copy(acc_sh.at[pl.ds(row0, WPT)],
                      p_out.at[c, pl.ds(row0, WPT)])
    @pl.when(s == NS - 1)
    def _():
      pltpu.sync_copy(acc_sh.at[pl.ds(row0, N - (NS - 1) * WPT)],
                      p_out.at[c, pl.ds(row0, N - (NS - 1) * WPT)])
    if with_deg:
      pltpu.sync_copy(deg_sh.at[pl.ds(row0, WPT)],
                      deg_out.at[c, pl.ds(row0, WPT)])

  return sc_agg


_sc_agg_deg = _make_sc_agg(True)
_sc_agg = _make_sc_agg(False)


GB = 1000
GRID = N // GB

_row_spec = pl.BlockSpec((GB, D), lambda i: (i, 0))
_w_spec = pl.BlockSpec((D, D), lambda i: (0, 0))
_b_spec = pl.BlockSpec((1, D), lambda i: (0, 0))
_p_spec = pl.BlockSpec((NC, GB, D), lambda i: (0, i, 0))
_dg_spec = pl.BlockSpec((NC, GB, 1), lambda i: (0, i, 0))


def _tc_entry_body(x_ref, wi_ref, bi_ref, ws_ref, b_ref, h_ref, s_ref):
  h = jnp.tanh(
      jnp.dot(x_ref[...], wi_ref[...], preferred_element_type=jnp.float32)
      + bi_ref[...])
  h_ref[...] = h
  s_ref[...] = (
      jnp.dot(h, ws_ref[...], preferred_element_type=jnp.float32) + b_ref[...])


_tc_entry = pl.pallas_call(
    _tc_entry_body,
    grid=(GRID,),
    in_specs=[_row_spec, _w_spec, _b_spec, _w_spec, _b_spec],
    out_specs=[_row_spec, _row_spec],
    out_shape=[jax.ShapeDtypeStruct((N, D), jnp.float32),
               jax.ShapeDtypeStruct((N, D), jnp.float32)])


def _neigh(p_ref, dg_ref, wn_ref):
  deg = dg_ref[0] + dg_ref[1]
  invd = 1.0 / jnp.maximum(deg, 1.0)
  h_neigh = (p_ref[0] + p_ref[1]) * invd
  return jnp.dot(h_neigh, wn_ref[...], preferred_element_type=jnp.float32)


def _tc_post_body(s_ref, p_ref, dg_ref, wn_ref, ws_ref, b_ref,
                  hn_ref, sn_ref):
  hn = jnp.maximum(s_ref[...] + _neigh(p_ref, dg_ref, wn_ref), 0.0)
  hn_ref[...] = hn
  sn_ref[...] = (
      jnp.dot(hn, ws_ref[...], preferred_element_type=jnp.float32) + b_ref[...])


_tc_post = pl.pallas_call(
    _tc_post_body,
    grid=(GRID,),
    in_specs=[_row_spec, _p_spec, _dg_spec, _w_spec, _w_spec, _b_spec],
    out_specs=[_row_spec, _row_spec],
    out_shape=[jax.ShapeDtypeStruct((N, D), jnp.float32),
               jax.ShapeDtypeStruct((N, D), jnp.float32)])


def _tc_last_body(s_ref, p_ref, dg_ref, wn_ref, hn_ref):
  hn_ref[...] = jnp.maximum(s_ref[...] + _neigh(p_ref, dg_ref, wn_ref), 0.0)


_tc_last = pl.pallas_call(
    _tc_last_body,
    grid=(GRID,),
    in_specs=[_row_spec, _p_spec, _dg_spec, _w_spec],
    out_specs=_row_spec,
    out_shape=jax.ShapeDtypeStruct((N, D), jnp.float32))


@jax.jit
def kernel(x, edge_index, W_in, b_in, W_self_0, W_neigh_0, b_0,
           W_self_1, W_neigh_1, b_1, W_self_2, W_neigh_2, b_2):
  ei = edge_index.astype(jnp.int32)
  pad = EP - E
  pad_iota = jnp.arange(pad, dtype=jnp.int32)
  srcp = jnp.concatenate([ei[0], (pad_iota * 131) % N])
  dstp = jnp.concatenate([ei[1], N + (pad_iota % (NP - N))])
  srcp = srcp.reshape(NC * NS, RPT, 128)
  dstp = dstp.reshape(NC * NS, RPT, 128)

  h, s = _tc_entry(x, W_in, b_in.reshape(1, D),
                   W_self_0, b_0.reshape(1, D))
  p, dg = _sc_agg_deg(h, srcp, dstp)
  dg3 = dg[:, :N, None]
  h, s = _tc_post(s, p, dg3, W_neigh_0, W_self_1, b_1.reshape(1, D))
  (p,) = _sc_agg(h, srcp, dstp)
  h, s = _tc_post(s, p, dg3, W_neigh_1, W_self_2, b_2.reshape(1, D))
  (p,) = _sc_agg(h, srcp, dstp)
  return _tc_last(s, p, dg3, W_neigh_2)


# speedup vs baseline: 1.0197x; 1.0197x over previous
import functools

import jax
import jax.numpy as jnp
from jax import lax
from jax.experimental import pallas as pl
from jax.experimental.pallas import tpu as pltpu
from jax.experimental.pallas import tpu_sc as plsc

N = 10000
NP = 10240
D = 128
E = 320000
EP = 327680
NC = 2
NS = 16
RPT = EP // (NC * NS * 128)
WPT = NP // NS


def _make_sc_agg(with_deg):
  out_type = [jax.ShapeDtypeStruct((NC, NP, D), jnp.float32)]
  if with_deg:
    out_type.append(jax.ShapeDtypeStruct((NC, NP), jnp.float32))
  mesh = plsc.VectorSubcoreMesh(
      core_axis_name="c", subcore_axis_name="s",
      num_cores=NC, num_subcores=NS)
  scratch = [
      pltpu.VMEM((2, 8, 128), jnp.int32),
      pltpu.VMEM((2, 8, 128), jnp.int32),
      pltpu.VMEM((4, 64, D), jnp.float32),
      pltpu.VMEM((64,), jnp.float32),
      pltpu.VMEM((WPT,), jnp.float32),
      pltpu.VMEM_SHARED((NP, D), jnp.float32),
      pltpu.VMEM_SHARED((NP,), jnp.float32),
      [pltpu.SemaphoreType.DMA] * 4,
      [pltpu.SemaphoreType.DMA] * 4,
      [pltpu.SemaphoreType.DMA] * 2,
  ]

  @functools.partial(pl.kernel, out_type=out_type, mesh=mesh,
                     scratch_types=scratch)
  def sc_agg(m_hbm, src_hbm, dst_hbm, *rest):
    if with_deg:
      (p_out, deg_out, src_v, dst_v, rows_v, ones_v, zrow_v, acc_sh, deg_sh,
       gsem, ssem, isem) = rest
    else:
      (p_out, src_v, dst_v, rows_v, ones_v, zrow_v, acc_sh, deg_sh,
       gsem, ssem, isem) = rest
    c = lax.axis_index("c")
    s = lax.axis_index("s")
    zero16 = jnp.zeros((16,), jnp.float32)

    def zrows(r, carry):
      for b in range(2):
        for q in range(D // 16):
          rows_v[b, r, pl.ds(q * 16, 16)] = zero16
      return carry
    lax.fori_loop(0, 64, zrows, 0)
    if with_deg:
      ones16 = jnp.ones((16,), jnp.float32)
      for q in range(64 // 16):
        ones_v[pl.ds(q * 16, 16)] = ones16
      def zdeg(i, carry):
        zrow_v[pl.ds(i * 16, 16)] = zero16
        return carry
      lax.fori_loop(0, WPT // 16, zdeg, 0)

    row0 = s * WPT
    def zacc(b, carry):
      pltpu.async_copy(rows_v.at[0], acc_sh.at[pl.ds(row0 + b * 128, 64)],
                       gsem[0])
      pltpu.async_copy(rows_v.at[1], acc_sh.at[pl.ds(row0 + b * 128 + 64, 64)],
                       gsem[1])
      return carry
    lax.fori_loop(0, WPT // 128, zacc, 0)
    def zacc_drain(b, carry):
      pltpu.make_async_copy(rows_v.at[0], acc_sh.at[pl.ds(row0, 64)],
                            gsem[0]).wait()
      pltpu.make_async_copy(rows_v.at[1], acc_sh.at[pl.ds(row0, 64)],
                            gsem[1]).wait()
      return carry
    lax.fori_loop(0, WPT // 128, zacc_drain, 0)
    if with_deg:
      pltpu.sync_copy(zrow_v, deg_sh.at[pl.ds(row0, WPT)])
    plsc.subcore_barrier()

    w = c * NS + s

    def idx_issue(t_next, dstset):
      start = pl.multiple_of(8 * t_next, 8)
      pltpu.async_copy(src_hbm.at[w, pl.ds(start, 8)], src_v.at[dstset],
                       isem[dstset])
      pltpu.async_copy(dst_hbm.at[w, pl.ds(start, 8)], dst_v.at[dstset],
                       isem[dstset])
    def idx_drain(dstset):
      pltpu.make_async_copy(src_hbm.at[w, pl.ds(0, 8)], src_v.at[dstset],
                            isem[dstset]).wait()
      pltpu.make_async_copy(dst_hbm.at[w, pl.ds(0, 8)], dst_v.at[dstset],
                            isem[dstset]).wait()
    def g_issue(P, u, b):
      pltpu.async_copy(
          m_hbm.at[src_v.at[P, u // 2, pl.ds((u % 2) * 64, 64)]],
          rows_v.at[b], gsem[b])
    def g_drain(b):
      pltpu.make_async_copy(m_hbm.at[src_v.at[0, 0, pl.ds(0, 64)]],
                            rows_v.at[b], gsem[b]).wait()
    def s_issue(P, u, b):
      pltpu.async_copy(rows_v.at[b],
                       acc_sh.at[dst_v.at[P, u // 2, pl.ds((u % 2) * 64, 64)]],
                       ssem[b], add=True)
      if with_deg:
        pltpu.sync_copy(ones_v,
                        deg_sh.at[dst_v.at[P, u // 2, pl.ds((u % 2) * 64, 64)]],
                        add=True)
    def s_drain(b):
      pltpu.make_async_copy(rows_v.at[b], acc_sh.at[dst_v.at[0, 0, pl.ds(0, 64)]],
                            ssem[b]).wait()

    def block(t_next, P, first, last):
      for u in range(16):
        b = u % 4
        b2 = (u + 2) % 4
        g_drain(b)
        s_issue(P, u, b)
        if u == 2 and not last:
          idx_issue(t_next, 1 - P)
        if not (first and u < 2):
          s_drain(b2)
        if u == 13 and not last:
          idx_drain(1 - P)
        if u < 14:
          g_issue(P, u + 2, b2)
        elif not last:
          g_issue(1 - P, u - 14, b2)

    idx_issue(jnp.int32(0), 0)
    idx_drain(0)
    g_issue(0, 0, 0)
    g_issue(0, 1, 1)
    block(jnp.int32(1), 0, first=True, last=False)

    def pair_body(p, carry):
      block(2 * p + 2, 1, first=False, last=False)
      block(2 * p + 3, 0, first=False, last=False)
      return carry
    lax.fori_loop(0, 4, pair_body, 0)

    block(jnp.int32(0), 1, first=False, last=True)
    s_drain(2)
    s_drain(3)
    plsc.subcore_barrier()

    pltpu.sync_copy(acc_sh.at[pl.ds(row0, WPT)], p_out.at[c, pl.ds(row0, WPT)])
    if with_deg:
      pltpu.sync_copy(deg_sh.at[pl.ds(row0, WPT)],
                      deg_out.at[c, pl.ds(row0, WPT)])

  return sc_agg


_sc_agg_deg = _make_sc_agg(True)
_sc_agg = _make_sc_agg(False)


def _tc_entry_body(x_ref, wi_ref, bi_ref, ws_ref, b_ref, h_ref, s_ref):
  h = jnp.tanh(
      jnp.dot(x_ref[...], wi_ref[...], preferred_element_type=jnp.float32)
      + bi_ref[...])
  h_ref[:N, :] = h
  s_ref[:N, :] = (
      jnp.dot(h, ws_ref[...], preferred_element_type=jnp.float32) + b_ref[...])


_tc_entry = pl.pallas_call(
    _tc_entry_body,
    out_shape=[jax.ShapeDtypeStruct((NP, D), jnp.float32),
               jax.ShapeDtypeStruct((NP, D), jnp.float32)])


def _neigh(p_ref, dg_ref, wn_ref):
  deg = dg_ref[0] + dg_ref[1]
  invd = 1.0 / jnp.maximum(deg, 1.0)
  h_neigh = (p_ref[0, :N] + p_ref[1, :N]) * invd[:N]
  return jnp.dot(h_neigh, wn_ref[...], preferred_element_type=jnp.float32)


def _tc_post_body(s_ref, p_ref, dg_ref, wn_ref, ws_ref, b_ref,
                  hn_ref, sn_ref):
  hn = jnp.maximum(s_ref[:N, :] + _neigh(p_ref, dg_ref, wn_ref), 0.0)
  hn_ref[:N, :] = hn
  sn_ref[:N, :] = (
      jnp.dot(hn, ws_ref[...], preferred_element_type=jnp.float32) + b_ref[...])


_tc_post = pl.pallas_call(
    _tc_post_body,
    out_shape=[jax.ShapeDtypeStruct((NP, D), jnp.float32),
               jax.ShapeDtypeStruct((NP, D), jnp.float32)])


def _tc_last_body(s_ref, p_ref, dg_ref, wn_ref, hn_ref):
  hn_ref[...] = jnp.maximum(s_ref[:N, :] + _neigh(p_ref, dg_ref, wn_ref), 0.0)


_tc_last = pl.pallas_call(
    _tc_last_body,
    out_shape=jax.ShapeDtypeStruct((N, D), jnp.float32))


@jax.jit
def kernel(x, edge_index, W_in, b_in, W_self_0, W_neigh_0, b_0,
           W_self_1, W_neigh_1, b_1, W_self_2, W_neigh_2, b_2):
  ei = edge_index.astype(jnp.int32)
  pad = EP - E
  pad_iota = jnp.arange(pad, dtype=jnp.int32)
  srcp = jnp.concatenate([ei[0], (pad_iota * 131) % N])
  dstp = jnp.concatenate([ei[1], N + (pad_iota % (NP - N))])
  srcp = srcp.reshape(NC * NS, RPT, 128)
  dstp = dstp.reshape(NC * NS, RPT, 128)

  h, s = _tc_entry(x, W_in, b_in.reshape(1, D),
                   W_self_0, b_0.reshape(1, D))
  p, dg = _sc_agg_deg(h, srcp, dstp)
  dg3 = dg[:, :, None]
  h, s = _tc_post(s, p, dg3, W_neigh_0, W_self_1, b_1.reshape(1, D))
  (p,) = _sc_agg(h, srcp, dstp)
  h, s = _tc_post(s, p, dg3, W_neigh_1, W_self_2, b_2.reshape(1, D))
  (p,) = _sc_agg(h, srcp, dstp)
  return _tc_last(s, p, dg3, W_neigh_2)
